# baseline (speedup 1.0000x reference)
import functools

import jax
import jax.numpy as jnp
from jax import lax
from jax.experimental import pallas as pl
from jax.experimental.pallas import tpu as pltpu
from jax.experimental.pallas import tpu_sc as plsc

B = 1024
NW = 32
SSPLIT = 256
SPW = (B - SSPLIT) // NW
TCHUNK = 1024
CHUNK = 256
NBUF = 3
LANES = 16
SPAD = 1088
RPAD = 1280


def _sc_segment_reduce(fv, starts_padded, n_rows, dv):
    nvec = dv // LANES
    mesh = plsc.VectorSubcoreMesh(core_axis_name="c", subcore_axis_name="s")

    def identity_accs():
        return (
            tuple(jnp.zeros((LANES,), jnp.float32) for _ in range(nvec)),
            tuple(jnp.full((LANES,), jnp.inf, jnp.float32) for _ in range(nvec)),
            tuple(jnp.full((LANES,), -jnp.inf, jnp.float32) for _ in range(nvec)),
        )

    @functools.partial(
        pl.kernel,
        out_type=[jax.ShapeDtypeStruct((B - SSPLIT, dv), jnp.float32)] * 3,
        mesh=mesh,
        scratch_types=[
            pltpu.VMEM((SPAD,), jnp.int32),
            pltpu.VMEM((CHUNK, dv), jnp.float32),
            pltpu.VMEM((CHUNK, dv), jnp.float32),
            pltpu.VMEM((CHUNK, dv), jnp.float32),
            pltpu.VMEM((SPW + 1, dv), jnp.float32),
            pltpu.VMEM((SPW + 1, dv), jnp.float32),
            pltpu.VMEM((SPW + 1, dv), jnp.float32),
            pltpu.SemaphoreType.DMA,
            pltpu.SemaphoreType.DMA,
            pltpu.SemaphoreType.DMA,
        ],
    )
    def body(fv_hbm, starts_hbm, sums_hbm, mins_hbm, maxs_hbm,
             starts_v, buf0_v, buf1_v, buf2_v, osum_v, omin_v, omax_v,
             sem0, sem1, sem2):
        wid = lax.axis_index("s") * 2 + lax.axis_index("c")
        pltpu.sync_copy(starts_hbm, starts_v)
        bufs = (buf0_v, buf1_v, buf2_v)
        sems = (sem0, sem1, sem2)

        seg0 = SSPLIT + wid * SPW
        r_first = starts_v[pl.ds(seg0, LANES)][0]
        r_last = starts_v[pl.ds(seg0 + SPW, LANES)][0]

        def count_bounds_le(x):
            lo = jnp.int32(0)
            sh = 1 << SPW.bit_length()
            while sh >= 1:
                cand = lo + sh
                bv = starts_v[pl.ds(seg0 + cand, LANES)][0]
                lo = jnp.where((cand <= SPW) & (bv <= x), cand, lo)
                sh //= 2
            return lo
        g0 = pl.multiple_of((r_first // 8) * 8, 8)
        nch = jnp.where(r_last > r_first, (r_last - g0 + CHUNK - 1) // CHUNK, 0)
        nch = ((nch + NBUF - 1) // NBUF) * NBUF

        def chunk_base(c):
            nom = g0 + c * CHUNK
            b0 = pl.multiple_of(jnp.minimum(nom, n_rows - CHUNK), 8)
            return nom, b0

        def start_dma(c, slot):
            _, b0 = chunk_base(c)
            pltpu.make_async_copy(
                fv_hbm.at[pl.ds(b0, CHUNK)], bufs[slot], sems[slot]).start()

        def wait_dma(c, slot):
            _, b0 = chunk_base(c)
            pltpu.make_async_copy(
                fv_hbm.at[pl.ds(b0, CHUNK)], bufs[slot], sems[slot]).wait()

        def next_boundary(seg):
            return starts_v[pl.ds(seg + 1, LANES)][0]

        def store_accs(local, accs):
            sums, mns, mxs = accs
            for j in range(nvec):
                osum_v[local, pl.ds(LANES * j, LANES)] = sums[j]
                omin_v[local, pl.ds(LANES * j, LANES)] = mns[j]
                omax_v[local, pl.ds(LANES * j, LANES)] = mxs[j]

        def process_chunk(c, slot, carry):
            wait_dma(c, slot)
            nom, b0 = chunk_base(c)
            r_hi = jnp.minimum(r_last, nom + CHUNK)
            buf = bufs[slot]

            def accum_row(o, accs):
                sums, mns, mxs = accs
                new_s, new_n, new_x = [], [], []
                for j in range(nvec):
                    v = buf[o, pl.ds(LANES * j, LANES)]
                    new_s.append(sums[j] + v)
                    new_n.append(jnp.minimum(mns[j], v))
                    new_x.append(jnp.maximum(mxs[j], v))
                return tuple(new_s), tuple(new_n), tuple(new_x)

            def wbody(_, st):
                r, seg, nb, accs = st
                active = r < r_hi
                seg_end = jnp.minimum(nb, r_hi)
                accs = lax.fori_loop(r - b0, seg_end - b0, accum_row, accs)
                store_accs(seg - seg0, accs)
                flag = (seg_end == nb) & active
                nb2 = next_boundary(seg + 1)
                accs = jax.tree.map(
                    lambda ident, a: jnp.where(flag, ident, a),
                    identity_accs(), accs)
                seg = jnp.where(flag, seg + 1, seg)
                nb = jnp.where(flag, nb2, nb)
                return seg_end, seg, nb, accs

            _, seg_in, _, _ = carry
            cnt = count_bounds_le(r_hi)
            trip = cnt - (seg_in - seg0) + 1
            r, seg, nb, accs = lax.fori_loop(0, trip, wbody, carry)

            @pl.when(c + NBUF < nch)
            def _():
                start_dma(c + NBUF, slot)

            return r, seg, nb, accs

        for b in range(NBUF):
            @pl.when(b < nch)
            def _(b=b):
                start_dma(jnp.int32(b), b)

        carry0 = (r_first, seg0, next_boundary(seg0), identity_accs())

        def group_body(g, carry):
            for b in range(NBUF):
                carry = process_chunk(g * NBUF + b, b, carry)
            return carry

        _, seg, _, accs = lax.fori_loop(0, nch // NBUF, group_body, carry0)

        store_accs(seg - seg0, accs)

        def tail_body(i, _):
            store_accs(i, identity_accs())
            return 0

        lax.fori_loop(jnp.minimum(seg - seg0 + 1, SPW), SPW, tail_body, 0)

        obase = pl.multiple_of(wid * SPW, 8)
        pltpu.sync_copy(osum_v.at[pl.ds(0, SPW)], sums_hbm.at[pl.ds(obase, SPW)])
        pltpu.sync_copy(omin_v.at[pl.ds(0, SPW)], mins_hbm.at[pl.ds(obase, SPW)])
        pltpu.sync_copy(omax_v.at[pl.ds(0, SPW)], maxs_hbm.at[pl.ds(obase, SPW)])

    return body(fv, starts_padded)


def _sc_find_starts(sid_padded, n_rows, slab, ng, sidpad):
    mesh = plsc.VectorSubcoreMesh(core_axis_name="c", subcore_axis_name="s")

    @functools.partial(
        pl.kernel,
        out_type=jax.ShapeDtypeStruct((NW * RPAD,), jnp.float32),
        mesh=mesh,
        scratch_types=[
            pltpu.VMEM((ng * LANES + LANES,), jnp.int32),
            pltpu.VMEM((RPAD,), jnp.float32),
        ],
        compiler_params=pltpu.CompilerParams(needs_layout_passes=False),
    )
    def body(sid_hbm, raw_hbm, slab_v, st_v):
        wid = lax.axis_index("s") * 2 + lax.axis_index("c")
        r0 = pl.multiple_of(wid * slab, 8)
        pltpu.sync_copy(sid_hbm.at[pl.ds(r0, ng * LANES + LANES)], slab_v)

        nfill = jnp.full((LANES,), n_rows, jnp.float32)

        def init_body(i, _):
            st_v[pl.ds(i * LANES, LANES)] = nfill
            return 0

        lax.fori_loop(0, RPAD // LANES, init_body, 0)

        iota = lax.iota(jnp.int32, LANES)

        def grp_body(g, _):
            v = slab_v[pl.ds(g * LANES, LANES)]
            vn = slab_v[pl.ds(g * LANES + 1, LANES)]
            val = ((r0 + g * LANES + 1) + iota).astype(jnp.float32)
            plsc.store_scatter(st_v, [vn], val, mask=vn != v)
            return 0

        lax.fori_loop(0, ng, grp_body, 0)
        obase = pl.multiple_of(wid * RPAD, 8)
        pltpu.sync_copy(st_v, raw_hbm.at[pl.ds(obase, RPAD)])

    return body(sid_padded)


def _tc_segment_reduce(fv, starts_padded, n_rows, dv):
    tnbuf = 2

    def body(starts_smem, fv_hbm, sums_ref, mins_ref, maxs_ref,
             buf0, buf1, sem0, sem1):
        bufs = (buf0, buf1)
        sems = (sem0, sem1)
        r_tc = starts_smem[SSPLIT]
        nch = jnp.where(r_tc > 0, (r_tc + TCHUNK - 1) // TCHUNK, 0)
        nch = ((nch + tnbuf - 1) // tnbuf) * tnbuf

        def chunk_base(c):
            nom = c * TCHUNK
            b0 = pl.multiple_of(jnp.minimum(nom, n_rows - TCHUNK), 8)
            return nom, b0

        def start_dma(c, slot):
            _, b0 = chunk_base(c)
            pltpu.make_async_copy(
                fv_hbm.at[pl.ds(b0, TCHUNK)], bufs[slot], sems[slot]).start()

        def wait_dma(c, slot):
            _, b0 = chunk_base(c)
            pltpu.make_async_copy(
                fv_hbm.at[pl.ds(b0, TCHUNK)], bufs[slot], sems[slot]).wait()

        T = 16

        def identity():
            return (jnp.zeros((T, dv), jnp.float32),
                    jnp.full((T, dv), jnp.inf, jnp.float32),
                    jnp.full((T, dv), -jnp.inf, jnp.float32))

        def flush(seg, asum, amin, amax):
            sums_ref[pl.ds(seg, 1), :] = jnp.sum(asum, axis=0, keepdims=True)
            mins_ref[pl.ds(seg, 1), :] = jnp.min(amin, axis=0, keepdims=True)
            maxs_ref[pl.ds(seg, 1), :] = jnp.max(amax, axis=0, keepdims=True)

        def process_chunk(c, slot, carry):
            wait_dma(c, slot)
            nom, b0 = chunk_base(c)
            r_hi = jnp.minimum(r_tc, nom + TCHUNK)
            buf = bufs[slot]

            def wbody(st):
                r, seg, nb, asum, amin, amax = st
                seg_end = jnp.minimum(nb, r_hi)
                a = r - b0
                e = seg_end - b0
                full_lo = (a + T - 1) // T
                full_hi = e // T
                head_end = jnp.minimum(e, full_lo * T)

                def masked_tile(lo, hi, kt, ac):
                    asum, amin, amax = ac
                    kt = jnp.clip(kt, 0, TCHUNK // T - 1)
                    v = buf[pl.ds(kt * T, T), :]
                    ri = kt * T + lax.broadcasted_iota(jnp.int32, (T, dv), 0)
                    m = (ri >= lo) & (ri < hi)
                    return (asum + jnp.where(m, v, 0.0),
                            jnp.minimum(amin, jnp.where(m, v, jnp.inf)),
                            jnp.maximum(amax, jnp.where(m, v, -jnp.inf)))

                def full_tile(kt, ac):
                    asum, amin, amax = ac
                    v = buf[pl.ds(kt * T, T), :]
                    return (asum + v, jnp.minimum(amin, v),
                            jnp.maximum(amax, v))

                ac = (asum, amin, amax)
                ac = masked_tile(a, head_end, a // T, ac)
                ac = masked_tile(jnp.maximum(full_hi * T, head_end), e,
                                 full_hi, ac)
                asum, amin, amax = lax.fori_loop(full_lo, full_hi, full_tile, ac)
                flag = seg_end == nb

                @pl.when(flag)
                def _():
                    flush(seg, asum, amin, amax)

                ident = identity()
                asum = jnp.where(flag, ident[0], asum)
                amin = jnp.where(flag, ident[1], amin)
                amax = jnp.where(flag, ident[2], amax)
                nb2 = starts_smem[jnp.minimum(seg + 2, B)]
                seg = jnp.where(flag, seg + 1, seg)
                nb = jnp.where(flag, nb2, nb)
                return seg_end, seg, nb, asum, amin, amax

            carry = lax.while_loop(lambda st: st[0] < r_hi, wbody, carry)

            @pl.when(c + tnbuf < nch)
            def _():
                start_dma(c + tnbuf, slot)

            return carry

        for b in range(tnbuf):
            @pl.when(b < nch)
            def _(b=b):
                start_dma(jnp.int32(b), b)

        carry = (jnp.int32(0), jnp.int32(0), starts_smem[1]) + identity()

        def group_body(g, carry):
            for b in range(tnbuf):
                carry = process_chunk(g * tnbuf + b, b, carry)
            return carry

        _, seg, _, asum, amin, amax = lax.fori_loop(
            0, nch // tnbuf, group_body, carry)

        @pl.when(seg < SSPLIT)
        def _():
            flush(seg, asum, amin, amax)

        def tail_body(i, _):
            flush(i, *identity())
            return 0

        lax.fori_loop(jnp.minimum(seg + 1, SSPLIT), SSPLIT, tail_body, 0)

    return pl.pallas_call(
        body,
        in_specs=[
            pl.BlockSpec(memory_space=pltpu.SMEM),
            pl.BlockSpec(memory_space=pl.ANY),
        ],
        out_shape=[jax.ShapeDtypeStruct((SSPLIT, dv), jnp.float32)] * 3,
        scratch_shapes=[
            pltpu.VMEM((TCHUNK, dv), jnp.float32),
            pltpu.VMEM((TCHUNK, dv), jnp.float32),
            pltpu.SemaphoreType.DMA,
            pltpu.SemaphoreType.DMA,
        ],
    )(starts_padded, fv)


def _tc_finish_body(sums_tc, sums_sc, mins_tc, mins_sc, maxs_tc, maxs_sc,
                    counts_ref, w1_ref, w2_ref, w3_ref, b_ref, out_ref):
    counts = counts_ref[:]
    inv = 1.0 / jnp.maximum(counts, 1.0)
    sums = jnp.concatenate([sums_tc[:], sums_sc[:]], axis=0)
    mins = jnp.concatenate([mins_tc[:], mins_sc[:]], axis=0)
    maxs = jnp.concatenate([maxs_tc[:], maxs_sc[:]], axis=0)
    mean = sums * inv
    mask = counts > 0.0
    mn = jnp.where(mask, mins, 0.0)
    mx = jnp.where(mask, maxs, 0.0)
    acc = jnp.dot(mean, w1_ref[:], preferred_element_type=jnp.float32)
    acc = acc + jnp.dot(mn, w2_ref[:], preferred_element_type=jnp.float32)
    acc = acc + jnp.dot(mx, w3_ref[:], preferred_element_type=jnp.float32)
    out_ref[:] = acc + b_ref[:]


def kernel(fv, segment_ids, num_segments, W, b):
    n_rows, dv = fv.shape
    dg = W.shape[1]
    shift = jnp.asarray(num_segments, jnp.int32) - B
    sid = segment_ids + shift

    slab = ((n_rows + NW - 1) // NW + LANES - 1) // LANES * LANES
    ng = slab // LANES + 1
    sidpad = (NW - 1) * slab + ng * LANES + LANES
    sid_padded = jnp.concatenate(
        [sid, jnp.full((sidpad - n_rows,), B, jnp.int32)])
    raw = _sc_find_starts(sid_padded, n_rows, slab, ng, sidpad)
    m = jnp.min(raw.reshape(NW, RPAD), axis=0)[: B + 1].astype(jnp.int32)
    bounds = jnp.arange(B + 1, dtype=jnp.int32)
    m = jnp.where(bounds <= sid[0], 0, m)
    k = 1
    while k <= B:
        m = jnp.minimum(
            m, jnp.concatenate([m[k:], jnp.full((k,), n_rows, jnp.int32)]))
        k *= 2
    starts = m
    starts_padded = jnp.full((SPAD,), n_rows, jnp.int32).at[: B + 1].set(starts)
    counts = (starts[1:] - starts[:-1]).astype(jnp.float32).reshape(B, 1)

    sums_sc, mins_sc, maxs_sc = _sc_segment_reduce(fv, starts_padded, n_rows, dv)
    sums_tc, mins_tc, maxs_tc = _tc_segment_reduce(fv, starts_padded, n_rows, dv)

    out = pl.pallas_call(
        _tc_finish_body,
        out_shape=jax.ShapeDtypeStruct((B, dg), jnp.float32),
    )(sums_tc, sums_sc, mins_tc, mins_sc, maxs_tc, maxs_sc, counts,
      W[0:dv, :], W[dv:2 * dv, :], W[2 * dv:3 * dv, :], b.reshape(1, dg))
    return out


# speedup vs baseline: 1.0541x; 1.0541x over previous
import functools

import jax
import jax.numpy as jnp
from jax import lax
from jax.experimental import pallas as pl
from jax.experimental.pallas import tpu as pltpu
from jax.experimental.pallas import tpu_sc as plsc

B = 1024
NW = 32
SSPLIT = 0
SPW = (B - SSPLIT) // NW
TCHUNK = 1024
CHUNK = 256
NBUF = 3
LANES = 16
SPAD = 1088
RPAD = 1280


def _sc_segment_reduce(fv, starts_padded, n_rows, dv):
    nvec = dv // LANES
    mesh = plsc.VectorSubcoreMesh(core_axis_name="c", subcore_axis_name="s")

    def identity_accs():
        return (
            tuple(jnp.zeros((LANES,), jnp.float32) for _ in range(nvec)),
            tuple(jnp.full((LANES,), jnp.inf, jnp.float32) for _ in range(nvec)),
            tuple(jnp.full((LANES,), -jnp.inf, jnp.float32) for _ in range(nvec)),
        )

    @functools.partial(
        pl.kernel,
        out_type=[jax.ShapeDtypeStruct((B - SSPLIT, dv), jnp.float32)] * 3,
        mesh=mesh,
        scratch_types=[
            pltpu.VMEM((SPAD,), jnp.int32),
            pltpu.VMEM((CHUNK, dv), jnp.float32),
            pltpu.VMEM((CHUNK, dv), jnp.float32),
            pltpu.VMEM((CHUNK, dv), jnp.float32),
            pltpu.VMEM((SPW + 1, dv), jnp.float32),
            pltpu.VMEM((SPW + 1, dv), jnp.float32),
            pltpu.VMEM((SPW + 1, dv), jnp.float32),
            pltpu.SemaphoreType.DMA,
            pltpu.SemaphoreType.DMA,
            pltpu.SemaphoreType.DMA,
        ],
    )
    def body(fv_hbm, starts_hbm, sums_hbm, mins_hbm, maxs_hbm,
             starts_v, buf0_v, buf1_v, buf2_v, osum_v, omin_v, omax_v,
             sem0, sem1, sem2):
        wid = lax.axis_index("s") * 2 + lax.axis_index("c")
        pltpu.sync_copy(starts_hbm, starts_v)
        bufs = (buf0_v, buf1_v, buf2_v)
        sems = (sem0, sem1, sem2)

        seg0 = SSPLIT + wid * SPW
        r_first = starts_v[pl.ds(seg0, LANES)][0]
        r_last = starts_v[pl.ds(seg0 + SPW, LANES)][0]

        def count_bounds_le(x):
            lo = jnp.int32(0)
            sh = 1 << SPW.bit_length()
            while sh >= 1:
                cand = lo + sh
                bv = starts_v[pl.ds(seg0 + cand, LANES)][0]
                lo = jnp.where((cand <= SPW) & (bv <= x), cand, lo)
                sh //= 2
            return lo
        g0 = pl.multiple_of((r_first // 8) * 8, 8)
        nch = jnp.where(r_last > r_first, (r_last - g0 + CHUNK - 1) // CHUNK, 0)
        nch = ((nch + NBUF - 1) // NBUF) * NBUF

        def chunk_base(c):
            nom = g0 + c * CHUNK
            b0 = pl.multiple_of(jnp.minimum(nom, n_rows - CHUNK), 8)
            return nom, b0

        def start_dma(c, slot):
            _, b0 = chunk_base(c)
            pltpu.make_async_copy(
                fv_hbm.at[pl.ds(b0, CHUNK)], bufs[slot], sems[slot]).start()

        def wait_dma(c, slot):
            _, b0 = chunk_base(c)
            pltpu.make_async_copy(
                fv_hbm.at[pl.ds(b0, CHUNK)], bufs[slot], sems[slot]).wait()

        def next_boundary(seg):
            return starts_v[pl.ds(seg + 1, LANES)][0]

        def store_accs(local, accs):
            sums, mns, mxs = accs
            for j in range(nvec):
                osum_v[local, pl.ds(LANES * j, LANES)] = sums[j]
                omin_v[local, pl.ds(LANES * j, LANES)] = mns[j]
                omax_v[local, pl.ds(LANES * j, LANES)] = mxs[j]

        def process_chunk(c, slot, carry):
            wait_dma(c, slot)
            nom, b0 = chunk_base(c)
            r_hi = jnp.minimum(r_last, nom + CHUNK)
            buf = bufs[slot]

            def accum_row(o, accs):
                sums, mns, mxs = accs
                new_s, new_n, new_x = [], [], []
                for j in range(nvec):
                    v = buf[o, pl.ds(LANES * j, LANES)]
                    new_s.append(sums[j] + v)
                    new_n.append(jnp.minimum(mns[j], v))
                    new_x.append(jnp.maximum(mxs[j], v))
                return tuple(new_s), tuple(new_n), tuple(new_x)

            def wbody(_, st):
                r, seg, nb, accs = st
                active = r < r_hi
                seg_end = jnp.minimum(nb, r_hi)
                accs = lax.fori_loop(r - b0, seg_end - b0, accum_row, accs)
                store_accs(seg - seg0, accs)
                flag = (seg_end == nb) & active
                nb2 = next_boundary(seg + 1)
                accs = jax.tree.map(
                    lambda ident, a: jnp.where(flag, ident, a),
                    identity_accs(), accs)
                seg = jnp.where(flag, seg + 1, seg)
                nb = jnp.where(flag, nb2, nb)
                return seg_end, seg, nb, accs

            _, seg_in, _, _ = carry
            cnt = count_bounds_le(r_hi)
            trip = cnt - (seg_in - seg0) + 1
            r, seg, nb, accs = lax.fori_loop(0, trip, wbody, carry)

            @pl.when(c + NBUF < nch)
            def _():
                start_dma(c + NBUF, slot)

            return r, seg, nb, accs

        for b in range(NBUF):
            @pl.when(b < nch)
            def _(b=b):
                start_dma(jnp.int32(b), b)

        carry0 = (r_first, seg0, next_boundary(seg0), identity_accs())

        def group_body(g, carry):
            for b in range(NBUF):
                carry = process_chunk(g * NBUF + b, b, carry)
            return carry

        _, seg, _, accs = lax.fori_loop(0, nch // NBUF, group_body, carry0)

        store_accs(seg - seg0, accs)

        def tail_body(i, _):
            store_accs(i, identity_accs())
            return 0

        lax.fori_loop(jnp.minimum(seg - seg0 + 1, SPW), SPW, tail_body, 0)

        obase = pl.multiple_of(wid * SPW, 8)
        pltpu.sync_copy(osum_v.at[pl.ds(0, SPW)], sums_hbm.at[pl.ds(obase, SPW)])
        pltpu.sync_copy(omin_v.at[pl.ds(0, SPW)], mins_hbm.at[pl.ds(obase, SPW)])
        pltpu.sync_copy(omax_v.at[pl.ds(0, SPW)], maxs_hbm.at[pl.ds(obase, SPW)])

    return body(fv, starts_padded)


def _sc_find_starts(sid_padded, n_rows, slab, ng, sidpad):
    mesh = plsc.VectorSubcoreMesh(core_axis_name="c", subcore_axis_name="s")

    @functools.partial(
        pl.kernel,
        out_type=jax.ShapeDtypeStruct((NW * RPAD,), jnp.float32),
        mesh=mesh,
        scratch_types=[
            pltpu.VMEM((ng * LANES + LANES,), jnp.int32),
            pltpu.VMEM((RPAD,), jnp.float32),
        ],
        compiler_params=pltpu.CompilerParams(needs_layout_passes=False),
    )
    def body(sid_hbm, raw_hbm, slab_v, st_v):
        wid = lax.axis_index("s") * 2 + lax.axis_index("c")
        r0 = pl.multiple_of(wid * slab, 8)
        pltpu.sync_copy(sid_hbm.at[pl.ds(r0, ng * LANES + LANES)], slab_v)

        nfill = jnp.full((LANES,), n_rows, jnp.float32)

        def init_body(i, _):
            st_v[pl.ds(i * LANES, LANES)] = nfill
            return 0

        lax.fori_loop(0, RPAD // LANES, init_body, 0)

        iota = lax.iota(jnp.int32, LANES)

        def grp_body(g, _):
            v = slab_v[pl.ds(g * LANES, LANES)]
            vn = slab_v[pl.ds(g * LANES + 1, LANES)]
            val = ((r0 + g * LANES + 1) + iota).astype(jnp.float32)
            plsc.store_scatter(st_v, [vn], val, mask=vn != v)
            return 0

        lax.fori_loop(0, ng, grp_body, 0)
        obase = pl.multiple_of(wid * RPAD, 8)
        pltpu.sync_copy(st_v, raw_hbm.at[pl.ds(obase, RPAD)])

    return body(sid_padded)


def _tc_segment_reduce(fv, starts_padded, n_rows, dv):
    tnbuf = 2

    def body(starts_smem, fv_hbm, sums_ref, mins_ref, maxs_ref,
             buf0, buf1, sem0, sem1):
        bufs = (buf0, buf1)
        sems = (sem0, sem1)
        r_tc = starts_smem[SSPLIT]
        nch = jnp.where(r_tc > 0, (r_tc + TCHUNK - 1) // TCHUNK, 0)
        nch = ((nch + tnbuf - 1) // tnbuf) * tnbuf

        def chunk_base(c):
            nom = c * TCHUNK
            b0 = pl.multiple_of(jnp.minimum(nom, n_rows - TCHUNK), 8)
            return nom, b0

        def start_dma(c, slot):
            _, b0 = chunk_base(c)
            pltpu.make_async_copy(
                fv_hbm.at[pl.ds(b0, TCHUNK)], bufs[slot], sems[slot]).start()

        def wait_dma(c, slot):
            _, b0 = chunk_base(c)
            pltpu.make_async_copy(
                fv_hbm.at[pl.ds(b0, TCHUNK)], bufs[slot], sems[slot]).wait()

        T = 32

        def identity():
            return (jnp.zeros((T, dv), jnp.float32),
                    jnp.full((T, dv), jnp.inf, jnp.float32),
                    jnp.full((T, dv), -jnp.inf, jnp.float32))

        def flush(seg, asum, amin, amax):
            sums_ref[pl.ds(seg, 1), :] = jnp.sum(asum, axis=0, keepdims=True)
            mins_ref[pl.ds(seg, 1), :] = jnp.min(amin, axis=0, keepdims=True)
            maxs_ref[pl.ds(seg, 1), :] = jnp.max(amax, axis=0, keepdims=True)

        def process_chunk(c, slot, carry):
            wait_dma(c, slot)
            nom, b0 = chunk_base(c)
            r_hi = jnp.minimum(r_tc, nom + TCHUNK)
            buf = bufs[slot]

            def wbody(st):
                r, seg, nb, asum, amin, amax = st
                seg_end = jnp.minimum(nb, r_hi)
                a = r - b0
                e = seg_end - b0
                full_lo = (a + T - 1) // T
                full_hi = e // T
                head_end = jnp.minimum(e, full_lo * T)

                def masked_tile(lo, hi, kt, ac):
                    asum, amin, amax = ac
                    kt = jnp.clip(kt, 0, TCHUNK // T - 1)
                    v = buf[pl.ds(kt * T, T), :]
                    ri = kt * T + lax.broadcasted_iota(jnp.int32, (T, dv), 0)
                    m = (ri >= lo) & (ri < hi)
                    return (asum + jnp.where(m, v, 0.0),
                            jnp.minimum(amin, jnp.where(m, v, jnp.inf)),
                            jnp.maximum(amax, jnp.where(m, v, -jnp.inf)))

                def full_tile(kt, ac):
                    asum, amin, amax = ac
                    v = buf[pl.ds(kt * T, T), :]
                    return (asum + v, jnp.minimum(amin, v),
                            jnp.maximum(amax, v))

                ac = (asum, amin, amax)
                ac = masked_tile(a, head_end, a // T, ac)
                ac = masked_tile(jnp.maximum(full_hi * T, head_end), e,
                                 full_hi, ac)
                asum, amin, amax = lax.fori_loop(full_lo, full_hi, full_tile, ac)
                flag = seg_end == nb

                @pl.when(flag)
                def _():
                    flush(seg, asum, amin, amax)

                ident = identity()
                asum = jnp.where(flag, ident[0], asum)
                amin = jnp.where(flag, ident[1], amin)
                amax = jnp.where(flag, ident[2], amax)
                nb2 = starts_smem[jnp.minimum(seg + 2, B)]
                seg = jnp.where(flag, seg + 1, seg)
                nb = jnp.where(flag, nb2, nb)
                return seg_end, seg, nb, asum, amin, amax

            carry = lax.while_loop(lambda st: st[0] < r_hi, wbody, carry)

            @pl.when(c + tnbuf < nch)
            def _():
                start_dma(c + tnbuf, slot)

            return carry

        for b in range(tnbuf):
            @pl.when(b < nch)
            def _(b=b):
                start_dma(jnp.int32(b), b)

        carry = (jnp.int32(0), jnp.int32(0), starts_smem[1]) + identity()

        def group_body(g, carry):
            for b in range(tnbuf):
                carry = process_chunk(g * tnbuf + b, b, carry)
            return carry

        _, seg, _, asum, amin, amax = lax.fori_loop(
            0, nch // tnbuf, group_body, carry)

        @pl.when(seg < SSPLIT)
        def _():
            flush(seg, asum, amin, amax)

        def tail_body(i, _):
            flush(i, *identity())
            return 0

        lax.fori_loop(jnp.minimum(seg + 1, SSPLIT), SSPLIT, tail_body, 0)

    return pl.pallas_call(
        body,
        in_specs=[
            pl.BlockSpec(memory_space=pltpu.SMEM),
            pl.BlockSpec(memory_space=pl.ANY),
        ],
        out_shape=[jax.ShapeDtypeStruct((SSPLIT, dv), jnp.float32)] * 3,
        scratch_shapes=[
            pltpu.VMEM((TCHUNK, dv), jnp.float32),
            pltpu.VMEM((TCHUNK, dv), jnp.float32),
            pltpu.SemaphoreType.DMA,
            pltpu.SemaphoreType.DMA,
        ],
    )(starts_padded, fv)


def _tc_finish_body(*refs):
    (sums_tc, sums_sc, mins_tc, mins_sc, maxs_tc, maxs_sc, counts_ref,
     w1_ref, w2_ref, w3_ref, b_ref, out_ref) = (
        refs if SSPLIT > 0 else
        (None, refs[0], None, refs[1], None, refs[2]) + refs[3:])
    counts = counts_ref[:]
    inv = 1.0 / jnp.maximum(counts, 1.0)

    def cat(tc, sc):
        return jnp.concatenate([tc[:], sc[:]], axis=0) if SSPLIT > 0 else sc[:]

    sums = cat(sums_tc, sums_sc)
    mins = cat(mins_tc, mins_sc)
    maxs = cat(maxs_tc, maxs_sc)
    mean = sums * inv
    mask = counts > 0.0
    mn = jnp.where(mask, mins, 0.0)
    mx = jnp.where(mask, maxs, 0.0)
    acc = jnp.dot(mean, w1_ref[:], preferred_element_type=jnp.float32)
    acc = acc + jnp.dot(mn, w2_ref[:], preferred_element_type=jnp.float32)
    acc = acc + jnp.dot(mx, w3_ref[:], preferred_element_type=jnp.float32)
    out_ref[:] = acc + b_ref[:]


def kernel(fv, segment_ids, num_segments, W, b):
    n_rows, dv = fv.shape
    dg = W.shape[1]
    shift = jnp.asarray(num_segments, jnp.int32) - B
    sid = segment_ids + shift

    slab = ((n_rows + NW - 1) // NW + LANES - 1) // LANES * LANES
    ng = slab // LANES + 1
    sidpad = (NW - 1) * slab + ng * LANES + LANES
    sid_padded = jnp.concatenate(
        [sid, jnp.full((sidpad - n_rows,), B, jnp.int32)])
    raw = _sc_find_starts(sid_padded, n_rows, slab, ng, sidpad)
    m = jnp.min(raw.reshape(NW, RPAD), axis=0)[: B + 1].astype(jnp.int32)
    bounds = jnp.arange(B + 1, dtype=jnp.int32)
    m = jnp.where(bounds <= sid[0], 0, m)
    k = 1
    while k <= B:
        m = jnp.minimum(
            m, jnp.concatenate([m[k:], jnp.full((k,), n_rows, jnp.int32)]))
        k *= 2
    starts = m
    starts_padded = jnp.full((SPAD,), n_rows, jnp.int32).at[: B + 1].set(starts)
    counts = (starts[1:] - starts[:-1]).astype(jnp.float32).reshape(B, 1)

    sums_sc, mins_sc, maxs_sc = _sc_segment_reduce(fv, starts_padded, n_rows, dv)
    if SSPLIT > 0:
        sums_tc, mins_tc, maxs_tc = _tc_segment_reduce(
            fv, starts_padded, n_rows, dv)
        pieces = (sums_tc, sums_sc, mins_tc, mins_sc, maxs_tc, maxs_sc)
    else:
        pieces = (sums_sc, mins_sc, maxs_sc)

    out = pl.pallas_call(
        _tc_finish_body,
        out_shape=jax.ShapeDtypeStruct((B, dg), jnp.float32),
    )(*pieces, counts,
      W[0:dv, :], W[dv:2 * dv, :], W[2 * dv:3 * dv, :], b.reshape(1, dg))
    return out


# speedup vs baseline: 1.0794x; 1.0240x over previous
import functools

import jax
import jax.numpy as jnp
from jax import lax
from jax.experimental import pallas as pl
from jax.experimental.pallas import tpu as pltpu
from jax.experimental.pallas import tpu_sc as plsc

B = 1024
NW = 32
SSPLIT = 0
SPW = (B - SSPLIT) // NW
TCHUNK = 1024
CHUNK = 256
NBUF = 3
LANES = 16
SPAD = 1088
RPAD = 1280


def _sc_segment_reduce(fv, starts_padded, n_rows, dv):
    nvec = dv // LANES
    mesh = plsc.VectorSubcoreMesh(core_axis_name="c", subcore_axis_name="s")

    def identity_accs():
        return (
            tuple(jnp.zeros((LANES,), jnp.float32) for _ in range(nvec)),
            tuple(jnp.full((LANES,), jnp.inf, jnp.float32) for _ in range(nvec)),
            tuple(jnp.full((LANES,), -jnp.inf, jnp.float32) for _ in range(nvec)),
        )

    @functools.partial(
        pl.kernel,
        out_type=[jax.ShapeDtypeStruct((B - SSPLIT, dv), jnp.float32)] * 3,
        mesh=mesh,
        scratch_types=[
            pltpu.VMEM((SPAD,), jnp.int32),
            pltpu.VMEM((CHUNK, dv), jnp.float32),
            pltpu.VMEM((CHUNK, dv), jnp.float32),
            pltpu.VMEM((CHUNK, dv), jnp.float32),
            pltpu.VMEM((SPW + 1, dv), jnp.float32),
            pltpu.VMEM((SPW + 1, dv), jnp.float32),
            pltpu.VMEM((SPW + 1, dv), jnp.float32),
            pltpu.SemaphoreType.DMA,
            pltpu.SemaphoreType.DMA,
            pltpu.SemaphoreType.DMA,
        ],
    )
    def body(fv_hbm, starts_hbm, sums_hbm, mins_hbm, maxs_hbm,
             starts_v, buf0_v, buf1_v, buf2_v, osum_v, omin_v, omax_v,
             sem0, sem1, sem2):
        wid = lax.axis_index("s") * 2 + lax.axis_index("c")
        pltpu.sync_copy(starts_hbm, starts_v)
        bufs = (buf0_v, buf1_v, buf2_v)
        sems = (sem0, sem1, sem2)

        seg0 = SSPLIT + wid * SPW
        r_first = starts_v[pl.ds(seg0, LANES)][0]
        r_last = starts_v[pl.ds(seg0 + SPW, LANES)][0]

        def count_bounds_le(x):
            lo = jnp.int32(0)
            sh = 1 << SPW.bit_length()
            while sh >= 1:
                cand = lo + sh
                bv = starts_v[pl.ds(seg0 + cand, LANES)][0]
                lo = jnp.where((cand <= SPW) & (bv <= x), cand, lo)
                sh //= 2
            return lo
        g0 = pl.multiple_of((r_first // 8) * 8, 8)
        nch = jnp.where(r_last > r_first, (r_last - g0 + CHUNK - 1) // CHUNK, 0)
        nch = ((nch + NBUF - 1) // NBUF) * NBUF

        def chunk_base(c):
            nom = g0 + c * CHUNK
            b0 = pl.multiple_of(jnp.minimum(nom, n_rows - CHUNK), 8)
            return nom, b0

        def start_dma(c, slot):
            _, b0 = chunk_base(c)
            pltpu.make_async_copy(
                fv_hbm.at[pl.ds(b0, CHUNK)], bufs[slot], sems[slot]).start()

        def wait_dma(c, slot):
            _, b0 = chunk_base(c)
            pltpu.make_async_copy(
                fv_hbm.at[pl.ds(b0, CHUNK)], bufs[slot], sems[slot]).wait()

        def next_boundary(seg):
            return starts_v[pl.ds(seg + 1, LANES)][0]

        def store_accs(local, accs):
            sums, mns, mxs = accs
            for j in range(nvec):
                osum_v[local, pl.ds(LANES * j, LANES)] = sums[j]
                omin_v[local, pl.ds(LANES * j, LANES)] = mns[j]
                omax_v[local, pl.ds(LANES * j, LANES)] = mxs[j]

        def process_chunk(c, slot, carry):
            wait_dma(c, slot)
            nom, b0 = chunk_base(c)
            r_hi = jnp.minimum(r_last, nom + CHUNK)
            buf = bufs[slot]

            def accum_row(o, accs):
                sums, mns, mxs = accs
                new_s, new_n, new_x = [], [], []
                for j in range(nvec):
                    v = buf[o, pl.ds(LANES * j, LANES)]
                    new_s.append(sums[j] + v)
                    new_n.append(jnp.minimum(mns[j], v))
                    new_x.append(jnp.maximum(mxs[j], v))
                return tuple(new_s), tuple(new_n), tuple(new_x)

            def wbody(_, st):
                r, seg, nb, accs = st
                active = r < r_hi
                seg_end = jnp.minimum(nb, r_hi)
                accs = lax.fori_loop(r - b0, seg_end - b0, accum_row, accs)
                store_accs(seg - seg0, accs)
                flag = (seg_end == nb) & active
                nb2 = next_boundary(seg + 1)
                accs = jax.tree.map(
                    lambda ident, a: jnp.where(flag, ident, a),
                    identity_accs(), accs)
                seg = jnp.where(flag, seg + 1, seg)
                nb = jnp.where(flag, nb2, nb)
                return seg_end, seg, nb, accs

            _, seg_in, _, _ = carry
            cnt = count_bounds_le(r_hi)
            trip = cnt - (seg_in - seg0) + 1
            r, seg, nb, accs = lax.fori_loop(0, trip, wbody, carry)

            @pl.when(c + NBUF < nch)
            def _():
                start_dma(c + NBUF, slot)

            return r, seg, nb, accs

        for b in range(NBUF):
            @pl.when(b < nch)
            def _(b=b):
                start_dma(jnp.int32(b), b)

        carry0 = (r_first, seg0, next_boundary(seg0), identity_accs())

        def group_body(g, carry):
            for b in range(NBUF):
                carry = process_chunk(g * NBUF + b, b, carry)
            return carry

        _, seg, _, accs = lax.fori_loop(0, nch // NBUF, group_body, carry0)

        store_accs(seg - seg0, accs)

        def tail_body(i, _):
            store_accs(i, identity_accs())
            return 0

        lax.fori_loop(jnp.minimum(seg - seg0 + 1, SPW), SPW, tail_body, 0)

        obase = pl.multiple_of(wid * SPW, 8)
        pltpu.sync_copy(osum_v.at[pl.ds(0, SPW)], sums_hbm.at[pl.ds(obase, SPW)])
        pltpu.sync_copy(omin_v.at[pl.ds(0, SPW)], mins_hbm.at[pl.ds(obase, SPW)])
        pltpu.sync_copy(omax_v.at[pl.ds(0, SPW)], maxs_hbm.at[pl.ds(obase, SPW)])

    return body(fv, starts_padded)


def _sc_find_starts(sid_padded, n_rows, slab, ng, sidpad):
    mesh = plsc.VectorSubcoreMesh(core_axis_name="c", subcore_axis_name="s")

    @functools.partial(
        pl.kernel,
        out_type=jax.ShapeDtypeStruct((NW * RPAD,), jnp.float32),
        mesh=mesh,
        scratch_types=[
            pltpu.VMEM((ng * LANES + LANES,), jnp.int32),
            pltpu.VMEM((RPAD,), jnp.float32),
        ],
        compiler_params=pltpu.CompilerParams(needs_layout_passes=False),
    )
    def body(sid_hbm, raw_hbm, slab_v, st_v):
        wid = lax.axis_index("s") * 2 + lax.axis_index("c")
        r0 = pl.multiple_of(wid * slab, 8)
        pltpu.sync_copy(sid_hbm.at[pl.ds(r0, ng * LANES + LANES)], slab_v)

        nfill = jnp.full((LANES,), n_rows, jnp.float32)

        def init_body(i, _):
            st_v[pl.ds(i * LANES, LANES)] = nfill
            return 0

        lax.fori_loop(0, RPAD // LANES, init_body, 0)

        iota = lax.iota(jnp.int32, LANES)

        def grp_body(g, _):
            v = slab_v[pl.ds(g * LANES, LANES)]
            vn = slab_v[pl.ds(g * LANES + 1, LANES)]
            val = ((r0 + g * LANES + 1) + iota).astype(jnp.float32)
            plsc.store_scatter(st_v, [vn], val, mask=vn != v)
            return 0

        lax.fori_loop(0, ng, grp_body, 0)
        obase = pl.multiple_of(wid * RPAD, 8)
        pltpu.sync_copy(st_v, raw_hbm.at[pl.ds(obase, RPAD)])

    return body(sid_padded)


def _tc_segment_reduce(fv, starts_padded, n_rows, dv):
    tnbuf = 2

    def body(starts_smem, fv_hbm, sums_ref, mins_ref, maxs_ref,
             buf0, buf1, sem0, sem1):
        bufs = (buf0, buf1)
        sems = (sem0, sem1)
        r_tc = starts_smem[SSPLIT]
        nch = jnp.where(r_tc > 0, (r_tc + TCHUNK - 1) // TCHUNK, 0)
        nch = ((nch + tnbuf - 1) // tnbuf) * tnbuf

        def chunk_base(c):
            nom = c * TCHUNK
            b0 = pl.multiple_of(jnp.minimum(nom, n_rows - TCHUNK), 8)
            return nom, b0

        def start_dma(c, slot):
            _, b0 = chunk_base(c)
            pltpu.make_async_copy(
                fv_hbm.at[pl.ds(b0, TCHUNK)], bufs[slot], sems[slot]).start()

        def wait_dma(c, slot):
            _, b0 = chunk_base(c)
            pltpu.make_async_copy(
                fv_hbm.at[pl.ds(b0, TCHUNK)], bufs[slot], sems[slot]).wait()

        T = 32

        def identity():
            return (jnp.zeros((T, dv), jnp.float32),
                    jnp.full((T, dv), jnp.inf, jnp.float32),
                    jnp.full((T, dv), -jnp.inf, jnp.float32))

        def flush(seg, asum, amin, amax):
            sums_ref[pl.ds(seg, 1), :] = jnp.sum(asum, axis=0, keepdims=True)
            mins_ref[pl.ds(seg, 1), :] = jnp.min(amin, axis=0, keepdims=True)
            maxs_ref[pl.ds(seg, 1), :] = jnp.max(amax, axis=0, keepdims=True)

        def process_chunk(c, slot, carry):
            wait_dma(c, slot)
            nom, b0 = chunk_base(c)
            r_hi = jnp.minimum(r_tc, nom + TCHUNK)
            buf = bufs[slot]

            def wbody(st):
                r, seg, nb, asum, amin, amax = st
                seg_end = jnp.minimum(nb, r_hi)
                a = r - b0
                e = seg_end - b0
                full_lo = (a + T - 1) // T
                full_hi = e // T
                head_end = jnp.minimum(e, full_lo * T)

                def masked_tile(lo, hi, kt, ac):
                    asum, amin, amax = ac
                    kt = jnp.clip(kt, 0, TCHUNK // T - 1)
                    v = buf[pl.ds(kt * T, T), :]
                    ri = kt * T + lax.broadcasted_iota(jnp.int32, (T, dv), 0)
                    m = (ri >= lo) & (ri < hi)
                    return (asum + jnp.where(m, v, 0.0),
                            jnp.minimum(amin, jnp.where(m, v, jnp.inf)),
                            jnp.maximum(amax, jnp.where(m, v, -jnp.inf)))

                def full_tile(kt, ac):
                    asum, amin, amax = ac
                    v = buf[pl.ds(kt * T, T), :]
                    return (asum + v, jnp.minimum(amin, v),
                            jnp.maximum(amax, v))

                ac = (asum, amin, amax)
                ac = masked_tile(a, head_end, a // T, ac)
                ac = masked_tile(jnp.maximum(full_hi * T, head_end), e,
                                 full_hi, ac)
                asum, amin, amax = lax.fori_loop(full_lo, full_hi, full_tile, ac)
                flag = seg_end == nb

                @pl.when(flag)
                def _():
                    flush(seg, asum, amin, amax)

                ident = identity()
                asum = jnp.where(flag, ident[0], asum)
                amin = jnp.where(flag, ident[1], amin)
                amax = jnp.where(flag, ident[2], amax)
                nb2 = starts_smem[jnp.minimum(seg + 2, B)]
                seg = jnp.where(flag, seg + 1, seg)
                nb = jnp.where(flag, nb2, nb)
                return seg_end, seg, nb, asum, amin, amax

            carry = lax.while_loop(lambda st: st[0] < r_hi, wbody, carry)

            @pl.when(c + tnbuf < nch)
            def _():
                start_dma(c + tnbuf, slot)

            return carry

        for b in range(tnbuf):
            @pl.when(b < nch)
            def _(b=b):
                start_dma(jnp.int32(b), b)

        carry = (jnp.int32(0), jnp.int32(0), starts_smem[1]) + identity()

        def group_body(g, carry):
            for b in range(tnbuf):
                carry = process_chunk(g * tnbuf + b, b, carry)
            return carry

        _, seg, _, asum, amin, amax = lax.fori_loop(
            0, nch // tnbuf, group_body, carry)

        @pl.when(seg < SSPLIT)
        def _():
            flush(seg, asum, amin, amax)

        def tail_body(i, _):
            flush(i, *identity())
            return 0

        lax.fori_loop(jnp.minimum(seg + 1, SSPLIT), SSPLIT, tail_body, 0)

    return pl.pallas_call(
        body,
        in_specs=[
            pl.BlockSpec(memory_space=pltpu.SMEM),
            pl.BlockSpec(memory_space=pl.ANY),
        ],
        out_shape=[jax.ShapeDtypeStruct((SSPLIT, dv), jnp.float32)] * 3,
        scratch_shapes=[
            pltpu.VMEM((TCHUNK, dv), jnp.float32),
            pltpu.VMEM((TCHUNK, dv), jnp.float32),
            pltpu.SemaphoreType.DMA,
            pltpu.SemaphoreType.DMA,
        ],
    )(starts_padded, fv)


def _tc_finish_body(*refs):
    (sums_tc, sums_sc, mins_tc, mins_sc, maxs_tc, maxs_sc, counts_ref,
     w1_ref, w2_ref, w3_ref, b_ref, out_ref) = (
        refs if SSPLIT > 0 else
        (None, refs[0], None, refs[1], None, refs[2]) + refs[3:])
    counts = counts_ref[:]
    inv = 1.0 / jnp.maximum(counts, 1.0)

    def cat(tc, sc):
        return jnp.concatenate([tc[:], sc[:]], axis=0) if SSPLIT > 0 else sc[:]

    sums = cat(sums_tc, sums_sc)
    mins = cat(mins_tc, mins_sc)
    maxs = cat(maxs_tc, maxs_sc)
    mean = sums * inv
    mask = counts > 0.0
    mn = jnp.where(mask, mins, 0.0)
    mx = jnp.where(mask, maxs, 0.0)
    acc = jnp.dot(mean, w1_ref[:], preferred_element_type=jnp.float32)
    acc = acc + jnp.dot(mn, w2_ref[:], preferred_element_type=jnp.float32)
    acc = acc + jnp.dot(mx, w3_ref[:], preferred_element_type=jnp.float32)
    out_ref[:] = acc + b_ref[:]


def kernel(fv, segment_ids, num_segments, W, b):
    n_rows, dv = fv.shape
    dg = W.shape[1]
    shift = jnp.asarray(num_segments, jnp.int32) - B
    sid = segment_ids + shift

    slab = ((n_rows + NW - 1) // NW + LANES - 1) // LANES * LANES
    ng = slab // LANES + 1
    sidpad = (NW - 1) * slab + ng * LANES + LANES
    sid_padded = jnp.concatenate(
        [sid, jnp.full((sidpad - n_rows,), B, jnp.int32)])
    raw = _sc_find_starts(sid_padded, n_rows, slab, ng, sidpad)
    m = jnp.min(raw.reshape(NW, RPAD), axis=0)[: B + 1].astype(jnp.int32)
    bounds = jnp.arange(B + 1, dtype=jnp.int32)
    m = jnp.where(bounds <= sid[0], 0, m)
    starts = lax.cummin(m[::-1])[::-1]
    starts_padded = jnp.full((SPAD,), n_rows, jnp.int32).at[: B + 1].set(starts)
    counts = (starts[1:] - starts[:-1]).astype(jnp.float32).reshape(B, 1)

    sums_sc, mins_sc, maxs_sc = _sc_segment_reduce(fv, starts_padded, n_rows, dv)
    if SSPLIT > 0:
        sums_tc, mins_tc, maxs_tc = _tc_segment_reduce(
            fv, starts_padded, n_rows, dv)
        pieces = (sums_tc, sums_sc, mins_tc, mins_sc, maxs_tc, maxs_sc)
    else:
        pieces = (sums_sc, mins_sc, maxs_sc)

    out = pl.pallas_call(
        _tc_finish_body,
        out_shape=jax.ShapeDtypeStruct((B, dg), jnp.float32),
    )(*pieces, counts,
      W[0:dv, :], W[dv:2 * dv, :], W[2 * dv:3 * dv, :], b.reshape(1, dg))
    return out


# speedup vs baseline: 1.1035x; 1.0224x over previous
import functools

import jax
import jax.numpy as jnp
from jax import lax
from jax.experimental import pallas as pl
from jax.experimental.pallas import tpu as pltpu
from jax.experimental.pallas import tpu_sc as plsc

B = 1024
NW = 32
SSPLIT = 0
SPW = (B - SSPLIT) // NW
TCHUNK = 1024
CHUNK = 288
NBUF = 3
LANES = 16
SPAD = 1088
RPAD = 1280


def _sc_segment_reduce(fv, starts_padded, n_rows, dv):
    nvec = dv // LANES
    mesh = plsc.VectorSubcoreMesh(core_axis_name="c", subcore_axis_name="s")

    def identity_accs():
        return (
            tuple(jnp.zeros((LANES,), jnp.float32) for _ in range(nvec)),
            tuple(jnp.full((LANES,), jnp.inf, jnp.float32) for _ in range(nvec)),
            tuple(jnp.full((LANES,), -jnp.inf, jnp.float32) for _ in range(nvec)),
        )

    @functools.partial(
        pl.kernel,
        out_type=[jax.ShapeDtypeStruct((B - SSPLIT, dv), jnp.float32)] * 3,
        mesh=mesh,
        scratch_types=[
            pltpu.VMEM((SPAD,), jnp.int32),
            pltpu.VMEM((CHUNK, dv), jnp.float32),
            pltpu.VMEM((CHUNK, dv), jnp.float32),
            pltpu.VMEM((CHUNK, dv), jnp.float32),
            pltpu.VMEM((SPW + 1, dv), jnp.float32),
            pltpu.VMEM((SPW + 1, dv), jnp.float32),
            pltpu.VMEM((SPW + 1, dv), jnp.float32),
            pltpu.SemaphoreType.DMA,
            pltpu.SemaphoreType.DMA,
            pltpu.SemaphoreType.DMA,
        ],
    )
    def body(fv_hbm, starts_hbm, sums_hbm, mins_hbm, maxs_hbm,
             starts_v, buf0_v, buf1_v, buf2_v, osum_v, omin_v, omax_v,
             sem0, sem1, sem2):
        wid = lax.axis_index("s") * 2 + lax.axis_index("c")
        pltpu.sync_copy(starts_hbm, starts_v)
        bufs = (buf0_v, buf1_v, buf2_v)
        sems = (sem0, sem1, sem2)

        seg0 = SSPLIT + wid * SPW
        r_first = starts_v[pl.ds(seg0, LANES)][0]
        r_last = starts_v[pl.ds(seg0 + SPW, LANES)][0]

        def count_bounds_le(x):
            lo = jnp.int32(0)
            sh = 1 << SPW.bit_length()
            while sh >= 1:
                cand = lo + sh
                bv = starts_v[pl.ds(seg0 + cand, LANES)][0]
                lo = jnp.where((cand <= SPW) & (bv <= x), cand, lo)
                sh //= 2
            return lo
        g0 = pl.multiple_of((r_first // 8) * 8, 8)
        nch = jnp.where(r_last > r_first, (r_last - g0 + CHUNK - 1) // CHUNK, 0)
        nch = ((nch + NBUF - 1) // NBUF) * NBUF

        def chunk_base(c):
            nom = g0 + c * CHUNK
            b0 = pl.multiple_of(jnp.minimum(nom, n_rows - CHUNK), 8)
            return nom, b0

        def start_dma(c, slot):
            _, b0 = chunk_base(c)
            pltpu.make_async_copy(
                fv_hbm.at[pl.ds(b0, CHUNK)], bufs[slot], sems[slot]).start()

        def wait_dma(c, slot):
            _, b0 = chunk_base(c)
            pltpu.make_async_copy(
                fv_hbm.at[pl.ds(b0, CHUNK)], bufs[slot], sems[slot]).wait()

        def next_boundary(seg):
            return starts_v[pl.ds(seg + 1, LANES)][0]

        def store_accs(local, accs):
            sums, mns, mxs = accs
            for j in range(nvec):
                osum_v[local, pl.ds(LANES * j, LANES)] = sums[j]
                omin_v[local, pl.ds(LANES * j, LANES)] = mns[j]
                omax_v[local, pl.ds(LANES * j, LANES)] = mxs[j]

        def process_chunk(c, slot, carry):
            wait_dma(c, slot)
            nom, b0 = chunk_base(c)
            r_hi = jnp.minimum(r_last, nom + CHUNK)
            buf = bufs[slot]

            def accum_row(o, accs):
                sums, mns, mxs = accs
                new_s, new_n, new_x = [], [], []
                for j in range(nvec):
                    v = buf[o, pl.ds(LANES * j, LANES)]
                    new_s.append(sums[j] + v)
                    new_n.append(jnp.minimum(mns[j], v))
                    new_x.append(jnp.maximum(mxs[j], v))
                return tuple(new_s), tuple(new_n), tuple(new_x)

            def wbody(_, st):
                r, seg, nb, accs = st
                active = r < r_hi
                seg_end = jnp.minimum(nb, r_hi)
                accs = lax.fori_loop(r - b0, seg_end - b0, accum_row, accs)
                store_accs(seg - seg0, accs)
                flag = (seg_end == nb) & active
                nb2 = next_boundary(seg + 1)
                accs = jax.tree.map(
                    lambda ident, a: jnp.where(flag, ident, a),
                    identity_accs(), accs)
                seg = jnp.where(flag, seg + 1, seg)
                nb = jnp.where(flag, nb2, nb)
                return seg_end, seg, nb, accs

            _, seg_in, _, _ = carry
            cnt = count_bounds_le(r_hi)
            trip = cnt - (seg_in - seg0) + 1
            r, seg, nb, accs = lax.fori_loop(0, trip, wbody, carry)

            @pl.when(c + NBUF < nch)
            def _():
                start_dma(c + NBUF, slot)

            return r, seg, nb, accs

        for b in range(NBUF):
            @pl.when(b < nch)
            def _(b=b):
                start_dma(jnp.int32(b), b)

        carry0 = (r_first, seg0, next_boundary(seg0), identity_accs())

        def group_body(g, carry):
            for b in range(NBUF):
                carry = process_chunk(g * NBUF + b, b, carry)
            return carry

        _, seg, _, accs = lax.fori_loop(0, nch // NBUF, group_body, carry0)

        store_accs(seg - seg0, accs)

        def tail_body(i, _):
            store_accs(i, identity_accs())
            return 0

        lax.fori_loop(jnp.minimum(seg - seg0 + 1, SPW), SPW, tail_body, 0)

        obase = pl.multiple_of(wid * SPW, 8)
        pltpu.sync_copy(osum_v.at[pl.ds(0, SPW)], sums_hbm.at[pl.ds(obase, SPW)])
        pltpu.sync_copy(omin_v.at[pl.ds(0, SPW)], mins_hbm.at[pl.ds(obase, SPW)])
        pltpu.sync_copy(omax_v.at[pl.ds(0, SPW)], maxs_hbm.at[pl.ds(obase, SPW)])

    return body(fv, starts_padded)


def _sc_find_starts(sid_padded, n_rows, slab, ng, sidpad):
    mesh = plsc.VectorSubcoreMesh(core_axis_name="c", subcore_axis_name="s")

    @functools.partial(
        pl.kernel,
        out_type=jax.ShapeDtypeStruct((NW * RPAD,), jnp.float32),
        mesh=mesh,
        scratch_types=[
            pltpu.VMEM((ng * LANES + LANES,), jnp.int32),
            pltpu.VMEM((RPAD,), jnp.float32),
        ],
        compiler_params=pltpu.CompilerParams(needs_layout_passes=False),
    )
    def body(sid_hbm, raw_hbm, slab_v, st_v):
        wid = lax.axis_index("s") * 2 + lax.axis_index("c")
        r0 = pl.multiple_of(wid * slab, 8)
        pltpu.sync_copy(sid_hbm.at[pl.ds(r0, ng * LANES + LANES)], slab_v)

        nfill = jnp.full((LANES,), n_rows, jnp.float32)

        def init_body(i, _):
            st_v[pl.ds(i * LANES, LANES)] = nfill
            return 0

        lax.fori_loop(0, RPAD // LANES, init_body, 0)

        iota = lax.iota(jnp.int32, LANES)

        def grp_body(g, _):
            v = slab_v[pl.ds(g * LANES, LANES)]
            vn = slab_v[pl.ds(g * LANES + 1, LANES)]
            val = ((r0 + g * LANES + 1) + iota).astype(jnp.float32)
            plsc.store_scatter(st_v, [vn], val, mask=vn != v)
            return 0

        lax.fori_loop(0, ng, grp_body, 0)
        obase = pl.multiple_of(wid * RPAD, 8)
        pltpu.sync_copy(st_v, raw_hbm.at[pl.ds(obase, RPAD)])

    return body(sid_padded)


def _tc_segment_reduce(fv, starts_padded, n_rows, dv):
    tnbuf = 2

    def body(starts_smem, fv_hbm, sums_ref, mins_ref, maxs_ref,
             buf0, buf1, sem0, sem1):
        bufs = (buf0, buf1)
        sems = (sem0, sem1)
        r_tc = starts_smem[SSPLIT]
        nch = jnp.where(r_tc > 0, (r_tc + TCHUNK - 1) // TCHUNK, 0)
        nch = ((nch + tnbuf - 1) // tnbuf) * tnbuf

        def chunk_base(c):
            nom = c * TCHUNK
            b0 = pl.multiple_of(jnp.minimum(nom, n_rows - TCHUNK), 8)
            return nom, b0

        def start_dma(c, slot):
            _, b0 = chunk_base(c)
            pltpu.make_async_copy(
                fv_hbm.at[pl.ds(b0, TCHUNK)], bufs[slot], sems[slot]).start()

        def wait_dma(c, slot):
            _, b0 = chunk_base(c)
            pltpu.make_async_copy(
                fv_hbm.at[pl.ds(b0, TCHUNK)], bufs[slot], sems[slot]).wait()

        T = 32

        def identity():
            return (jnp.zeros((T, dv), jnp.float32),
                    jnp.full((T, dv), jnp.inf, jnp.float32),
                    jnp.full((T, dv), -jnp.inf, jnp.float32))

        def flush(seg, asum, amin, amax):
            sums_ref[pl.ds(seg, 1), :] = jnp.sum(asum, axis=0, keepdims=True)
            mins_ref[pl.ds(seg, 1), :] = jnp.min(amin, axis=0, keepdims=True)
            maxs_ref[pl.ds(seg, 1), :] = jnp.max(amax, axis=0, keepdims=True)

        def process_chunk(c, slot, carry):
            wait_dma(c, slot)
            nom, b0 = chunk_base(c)
            r_hi = jnp.minimum(r_tc, nom + TCHUNK)
            buf = bufs[slot]

            def wbody(st):
                r, seg, nb, asum, amin, amax = st
                seg_end = jnp.minimum(nb, r_hi)
                a = r - b0
                e = seg_end - b0
                full_lo = (a + T - 1) // T
                full_hi = e // T
                head_end = jnp.minimum(e, full_lo * T)

                def masked_tile(lo, hi, kt, ac):
                    asum, amin, amax = ac
                    kt = jnp.clip(kt, 0, TCHUNK // T - 1)
                    v = buf[pl.ds(kt * T, T), :]
                    ri = kt * T + lax.broadcasted_iota(jnp.int32, (T, dv), 0)
                    m = (ri >= lo) & (ri < hi)
                    return (asum + jnp.where(m, v, 0.0),
                            jnp.minimum(amin, jnp.where(m, v, jnp.inf)),
                            jnp.maximum(amax, jnp.where(m, v, -jnp.inf)))

                def full_tile(kt, ac):
                    asum, amin, amax = ac
                    v = buf[pl.ds(kt * T, T), :]
                    return (asum + v, jnp.minimum(amin, v),
                            jnp.maximum(amax, v))

                ac = (asum, amin, amax)
                ac = masked_tile(a, head_end, a // T, ac)
                ac = masked_tile(jnp.maximum(full_hi * T, head_end), e,
                                 full_hi, ac)
                asum, amin, amax = lax.fori_loop(full_lo, full_hi, full_tile, ac)
                flag = seg_end == nb

                @pl.when(flag)
                def _():
                    flush(seg, asum, amin, amax)

                ident = identity()
                asum = jnp.where(flag, ident[0], asum)
                amin = jnp.where(flag, ident[1], amin)
                amax = jnp.where(flag, ident[2], amax)
                nb2 = starts_smem[jnp.minimum(seg + 2, B)]
                seg = jnp.where(flag, seg + 1, seg)
                nb = jnp.where(flag, nb2, nb)
                return seg_end, seg, nb, asum, amin, amax

            carry = lax.while_loop(lambda st: st[0] < r_hi, wbody, carry)

            @pl.when(c + tnbuf < nch)
            def _():
                start_dma(c + tnbuf, slot)

            return carry

        for b in range(tnbuf):
            @pl.when(b < nch)
            def _(b=b):
                start_dma(jnp.int32(b), b)

        carry = (jnp.int32(0), jnp.int32(0), starts_smem[1]) + identity()

        def group_body(g, carry):
            for b in range(tnbuf):
                carry = process_chunk(g * tnbuf + b, b, carry)
            return carry

        _, seg, _, asum, amin, amax = lax.fori_loop(
            0, nch // tnbuf, group_body, carry)

        @pl.when(seg < SSPLIT)
        def _():
            flush(seg, asum, amin, amax)

        def tail_body(i, _):
            flush(i, *identity())
            return 0

        lax.fori_loop(jnp.minimum(seg + 1, SSPLIT), SSPLIT, tail_body, 0)

    return pl.pallas_call(
        body,
        in_specs=[
            pl.BlockSpec(memory_space=pltpu.SMEM),
            pl.BlockSpec(memory_space=pl.ANY),
        ],
        out_shape=[jax.ShapeDtypeStruct((SSPLIT, dv), jnp.float32)] * 3,
        scratch_shapes=[
            pltpu.VMEM((TCHUNK, dv), jnp.float32),
            pltpu.VMEM((TCHUNK, dv), jnp.float32),
            pltpu.SemaphoreType.DMA,
            pltpu.SemaphoreType.DMA,
        ],
    )(starts_padded, fv)


def _tc_finish_body(*refs):
    (sums_tc, sums_sc, mins_tc, mins_sc, maxs_tc, maxs_sc, counts_ref,
     w1_ref, w2_ref, w3_ref, b_ref, out_ref) = (
        refs if SSPLIT > 0 else
        (None, refs[0], None, refs[1], None, refs[2]) + refs[3:])
    counts = counts_ref[:]
    inv = 1.0 / jnp.maximum(counts, 1.0)

    def cat(tc, sc):
        return jnp.concatenate([tc[:], sc[:]], axis=0) if SSPLIT > 0 else sc[:]

    sums = cat(sums_tc, sums_sc)
    mins = cat(mins_tc, mins_sc)
    maxs = cat(maxs_tc, maxs_sc)
    mean = sums * inv
    mask = counts > 0.0
    mn = jnp.where(mask, mins, 0.0)
    mx = jnp.where(mask, maxs, 0.0)
    acc = jnp.dot(mean, w1_ref[:], preferred_element_type=jnp.float32)
    acc = acc + jnp.dot(mn, w2_ref[:], preferred_element_type=jnp.float32)
    acc = acc + jnp.dot(mx, w3_ref[:], preferred_element_type=jnp.float32)
    out_ref[:] = acc + b_ref[:]


def kernel(fv, segment_ids, num_segments, W, b):
    n_rows, dv = fv.shape
    dg = W.shape[1]
    shift = jnp.asarray(num_segments, jnp.int32) - B
    sid = segment_ids + shift

    slab = ((n_rows + NW - 1) // NW + LANES - 1) // LANES * LANES
    ng = slab // LANES + 1
    sidpad = (NW - 1) * slab + ng * LANES + LANES
    sid_padded = jnp.concatenate(
        [sid, jnp.full((sidpad - n_rows,), B, jnp.int32)])
    raw = _sc_find_starts(sid_padded, n_rows, slab, ng, sidpad)
    m = jnp.min(raw.reshape(NW, RPAD), axis=0)[: B + 1].astype(jnp.int32)
    bounds = jnp.arange(B + 1, dtype=jnp.int32)
    m = jnp.where(bounds <= sid[0], 0, m)
    starts = lax.cummin(m[::-1])[::-1]
    starts_padded = jnp.full((SPAD,), n_rows, jnp.int32).at[: B + 1].set(starts)
    counts = (starts[1:] - starts[:-1]).astype(jnp.float32).reshape(B, 1)

    sums_sc, mins_sc, maxs_sc = _sc_segment_reduce(fv, starts_padded, n_rows, dv)
    if SSPLIT > 0:
        sums_tc, mins_tc, maxs_tc = _tc_segment_reduce(
            fv, starts_padded, n_rows, dv)
        pieces = (sums_tc, sums_sc, mins_tc, mins_sc, maxs_tc, maxs_sc)
    else:
        pieces = (sums_sc, mins_sc, maxs_sc)

    out = pl.pallas_call(
        _tc_finish_body,
        out_shape=jax.ShapeDtypeStruct((B, dg), jnp.float32),
    )(*pieces, counts,
      W[0:dv, :], W[dv:2 * dv, :], W[2 * dv:3 * dv, :], b.reshape(1, dg))
    return out
